# baseline (speedup 1.0000x reference)
import jax
import jax.numpy as jnp
from jax.experimental import pallas as pl
from jax.experimental.pallas import tpu as pltpu

_N, _M, _D = 4096, 4096, 64
_BR = 256
_G = _BR // 8
_LOG2E = 1.4426950408889634


def _matcher_kernel(cd_ref, d1_ref, d2_ref, out_ref,
                    d1b_ref, d2b_ref, rn_ref, cn_ref, cs_ref, ci_ref):
    p = pl.program_id(0)
    i = pl.program_id(1)
    cd = cd_ref[0, 0]

    @pl.when(jnp.logical_and(p == 0, i == 0))
    def _init():
        d1 = d1_ref[...]
        d2 = d2_ref[...]
        d1b_ref[...] = d1.astype(jnp.bfloat16)
        d2b_ref[...] = (-2.0 * d2).astype(jnp.bfloat16)
        rn_ref[...] = jnp.sum(d1 * d1, axis=1).reshape(_N // 8, 8)
        cn = jnp.sum(d2 * d2, axis=1)[None, :] + 1e-12
        cn_ref[...] = jnp.broadcast_to(cn, (8, _M))
        cs_ref[...] = jnp.zeros_like(cs_ref)

    ab = jax.lax.dot_general(
        d1b_ref[pl.ds(i * _BR, _BR), :], d2b_ref[...],
        (((1,), (1,)), ((), ())),
        preferred_element_type=jnp.float32)
    rn3 = rn_ref[pl.ds(i * _G, _G), :].reshape(_G, 8, 1)
    sq3 = jnp.maximum(rn3 + cn_ref[...][None] + ab.reshape(_G, 8, _M),
                      1e-12)
    arg = (cd * sq3) * jax.lax.rsqrt(sq3)

    @pl.when(p == 0)
    def _stats():
        cs_ref[...] += jnp.sum(jnp.exp2(arg), axis=0)

    @pl.when(jnp.logical_and(p == 1, i == 0))
    def _colinv():
        tot = jnp.sum(cs_ref[...], axis=0, keepdims=True)
        ci_ref[...] = jnp.broadcast_to(1.0 / tot, (8, _M))

    @pl.when(p == 1)
    def _emit():
        e3 = jnp.exp2(arg)
        rinv3 = (1.0 / jnp.sum(e3, axis=2)).reshape(_G, 8, 1)
        out_ref[...] = ((e3 * e3) * rinv3 * ci_ref[...][None]
                        ).reshape(_BR, _M)


def kernel(desc_1, desc_2, inverse_T):
    cd = jnp.reshape(-inverse_T.astype(jnp.float32) * _LOG2E, (1, 1))
    nb = _N // _BR
    return pl.pallas_call(
        _matcher_kernel,
        grid=(2, nb),
        in_specs=[
            pl.BlockSpec(memory_space=pltpu.SMEM),
            pl.BlockSpec((_N, _D), lambda p, i: (0, 0)),
            pl.BlockSpec((_M, _D), lambda p, i: (0, 0)),
        ],
        out_specs=pl.BlockSpec((_BR, _M), lambda p, i: (p * i, 0)),
        out_shape=jax.ShapeDtypeStruct((_N, _M), jnp.float32),
        scratch_shapes=[
            pltpu.VMEM((_N, _D), jnp.bfloat16),
            pltpu.VMEM((_M, _D), jnp.bfloat16),
            pltpu.VMEM((_N // 8, 8), jnp.float32),
            pltpu.VMEM((8, _M), jnp.float32),
            pltpu.VMEM((8, _M), jnp.float32),
            pltpu.VMEM((8, _M), jnp.float32),
        ],
        compiler_params=pltpu.CompilerParams(
            dimension_semantics=("arbitrary", "arbitrary")),
    )(cd, desc_1.astype(jnp.float32), desc_2.astype(jnp.float32))


# speedup vs baseline: 1.1138x; 1.1138x over previous
import jax
import jax.numpy as jnp
from jax.experimental import pallas as pl
from jax.experimental.pallas import tpu as pltpu

_N, _M, _D = 4096, 4096, 64
_K = _D + 2
_BR = 256
_G = _BR // 8
_LOG2E = 1.4426950408889634


def _matcher_kernel(cd_ref, d1_ref, d2_ref, out_ref,
                    d1a_ref, d2a_ref, cs_ref, ci_ref):
    p = pl.program_id(0)
    i = pl.program_id(1)
    cd = cd_ref[0, 0]

    @pl.when(jnp.logical_and(p == 0, i == 0))
    def _init():
        d1 = d1_ref[...]
        d2 = d2_ref[...]
        d1a_ref[:, 0:_D] = d1
        d1a_ref[:, _D:_D + 1] = jnp.sum(d1 * d1, axis=1, keepdims=True)
        d1a_ref[:, _D + 1:_K] = jnp.ones((_N, 1), jnp.float32)
        d2a_ref[:, 0:_D] = -2.0 * d2
        d2a_ref[:, _D:_D + 1] = jnp.ones((_M, 1), jnp.float32)
        d2a_ref[:, _D + 1:_K] = (jnp.sum(d2 * d2, axis=1, keepdims=True)
                                 + 1e-12)
        cs_ref[...] = jnp.zeros_like(cs_ref)

    _H = _BR // 2

    def _half_arg(h):
        sq = jax.lax.dot_general(
            d1a_ref[pl.ds(i * _BR + h * _H, _H), :], d2a_ref[...],
            (((1,), (1,)), ((), ())),
            preferred_element_type=jnp.float32)
        sq3 = jnp.maximum(sq.reshape(_G // 2, 8, _M), 1e-12)
        return (cd * sq3) * jax.lax.rsqrt(sq3)

    arg_a = _half_arg(0)
    arg_b = _half_arg(1)

    @pl.when(p == 0)
    def _stats():
        cs_ref[...] += (jnp.sum(jnp.exp2(arg_a), axis=0)
                        + jnp.sum(jnp.exp2(arg_b), axis=0))

    @pl.when(jnp.logical_and(p == 1, i == 0))
    def _colinv():
        tot = jnp.sum(cs_ref[...], axis=0, keepdims=True)
        ci_ref[...] = jnp.broadcast_to(1.0 / tot, (8, _M))

    @pl.when(p == 1)
    def _emit():
        ci3 = ci_ref[...][None]
        for h, arg in ((0, arg_a), (1, arg_b)):
            e3 = jnp.exp2(arg)
            rinv3 = (1.0 / jnp.sum(e3, axis=2)).reshape(_G // 2, 8, 1)
            out_ref[pl.ds(h * _H, _H), :] = (
                (e3 * e3) * rinv3 * ci3).reshape(_H, _M)


def kernel(desc_1, desc_2, inverse_T):
    cd = jnp.reshape(-inverse_T.astype(jnp.float32) * _LOG2E, (1, 1))
    nb = _N // _BR
    return pl.pallas_call(
        _matcher_kernel,
        grid=(2, nb),
        in_specs=[
            pl.BlockSpec(memory_space=pltpu.SMEM),
            pl.BlockSpec((_N, _D), lambda p, i: (0, 0)),
            pl.BlockSpec((_M, _D), lambda p, i: (0, 0)),
        ],
        out_specs=pl.BlockSpec((_BR, _M), lambda p, i: (p * i, 0)),
        out_shape=jax.ShapeDtypeStruct((_N, _M), jnp.float32),
        scratch_shapes=[
            pltpu.VMEM((_N, _K), jnp.float32),
            pltpu.VMEM((_M, _K), jnp.float32),
            pltpu.VMEM((8, _M), jnp.float32),
            pltpu.VMEM((8, _M), jnp.float32),
        ],
        compiler_params=pltpu.CompilerParams(
            dimension_semantics=("arbitrary", "arbitrary")),
    )(cd, desc_1.astype(jnp.float32), desc_2.astype(jnp.float32))
